# Initial kernel scaffold; baseline (speedup 1.0000x reference)
import jax
import jax.numpy as jnp
from jax.experimental import pallas as pl


def kernel(patch_size, feats, num_patches, patch_ids, patch_local_ids, W1, b1, gamma1, beta1, W2, b2):
    raise NotImplementedError("write your pallas kernel here")


# speedup vs baseline: 14.0819x; 14.0819x over previous
import functools

import jax
import jax.numpy as jnp
from jax import lax
from jax.experimental import pallas as pl
from jax.experimental.pallas import tpu as pltpu
from jax.experimental.pallas import tpu_sc as plsc

B = 8
HW = 1024
C = 512
N = 512
K = 48
KTOP = 24
ROWS = B * N
NC, NS = 2, 16
NW = NC * NS
RPW = ROWS // NW

_F32 = jnp.float32
_I32 = jnp.int32
_HIGHEST = lax.Precision.HIGHEST


def _gram_body(f_ref, s_ref):
    x = f_ref[...]
    ss = jnp.sum(x * x, axis=1, keepdims=True)
    nrm = jnp.maximum(jnp.sqrt(ss), 1e-12)
    xn = x / nrm
    s_ref[...] = lax.dot_general(
        xn, xn, (((1,), (1,)), ((), ())),
        preferred_element_type=_F32)


def _gram(feat_r):
    return pl.pallas_call(
        _gram_body,
        grid=(B,),
        in_specs=[pl.BlockSpec((None, HW, C), lambda i: (i, 0, 0))],
        out_specs=pl.BlockSpec((None, HW, HW), lambda i: (i, 0, 0)),
        out_shape=jax.ShapeDtypeStruct((B, HW, HW), _F32),
    )(feat_r)


_SUBCH = 32


def _simgather_body(s_ref, pid_ref, lidT_ref, simT_ref,
                    lT, pref, idxb, rows_v, sbufT, sem):
    wid = lax.axis_index("s") * NC + lax.axis_index("c")
    b = wid // 4
    n0 = (wid % 4) * 128
    row0 = wid * RPW
    pltpu.sync_copy(lidT_ref.at[:, pl.ds(n0, 128)], lT)
    pltpu.sync_copy(pid_ref.at[pl.ds(n0, 128)], pref)
    iota16 = lax.iota(_I32, 16)

    for sc in range(RPW // _SUBCH):
        for h in range(2):
            idxb[pl.ds(16 * h, 16)] = pref[pl.ds(sc * _SUBCH + 16 * h, 16)] \
                + b * HW
        pltpu.async_copy(s_ref.at[idxb], rows_v, sem).wait()
        for g in range(2):
            maj = iota16 + g * 16

            def inner(j, _, _sc=sc, _g=g, _maj=maj):
                col = _sc * _SUBCH + _g * 16
                lvec = lT[j, pl.ds(col, 16)]
                sbufT[j, pl.ds(col, 16)] = plsc.load_gather(
                    rows_v, [_maj, lvec])
                return 0

            lax.fori_loop(0, K, inner, 0)
    pltpu.sync_copy(sbufT, simT_ref.at[:, pl.ds(row0, RPW)])


def _simgather(s_rows, patch_id, local_id_T):
    mesh = plsc.VectorSubcoreMesh(
        core_axis_name="c", subcore_axis_name="s",
        num_cores=NC, num_subcores=NS)
    fn = pl.kernel(
        _simgather_body,
        out_type=jax.ShapeDtypeStruct((K, ROWS), _F32),
        mesh=mesh,
        compiler_params=pltpu.CompilerParams(needs_layout_passes=False),
        scratch_types=[
            pltpu.VMEM((K, 128), _I32),
            pltpu.VMEM((128,), _I32),
            pltpu.VMEM((_SUBCH,), _I32),
            pltpu.VMEM((_SUBCH, HW), _F32),
            pltpu.VMEM((K, 128), _F32),
            pltpu.SemaphoreType.DMA,
        ],
    )
    return fn(s_rows, patch_id, local_id_T)


def _topk_body(sim_ref, loc_ref, pid_ref, top_ref, pos_ref, w_ref):
    s = jnp.transpose(sim_ref[...])
    loc = loc_ref[...]
    lane = lax.broadcasted_iota(_I32, (N, K), 1)
    rank = jnp.zeros((N, K), _I32)
    for j in range(K):
        sj = s[:, j:j + 1]
        beats = (sj > s) | ((sj == s) & (lane > j))
        rank = rank + beats.astype(_I32)

    top_cols, sel_cols = [], []
    for r in range(KTOP):
        m = rank == r
        top_cols.append(jnp.sum(jnp.where(m, lane, 0), axis=1, keepdims=True))
        sel_cols.append(jnp.sum(jnp.where(m, loc, 0), axis=1, keepdims=True))
    top_ref[...] = jnp.concatenate(top_cols, axis=1)

    arr = jnp.concatenate([pid_ref[...]] + sel_cols, axis=1)
    lane25 = lax.broadcasted_iota(_I32, (N, KTOP + 1), 1)
    w_cols = []
    for j in range(KTOP + 1):
        aj = arr[:, j:j + 1]
        eq = (aj == arr)
        cnt = jnp.sum(eq.astype(_F32), axis=1, keepdims=True)
        before = jnp.sum((eq & (lane25 < j)).astype(_I32), axis=1,
                         keepdims=True)
        w_cols.append(jnp.where(before == 0, cnt, 0.0))
    pad_i = jnp.zeros((N, 32 - (KTOP + 1)), _I32)
    pad_f = jnp.zeros((N, 32 - (KTOP + 1)), _F32)
    pos_ref[...] = jnp.concatenate([arr, pad_i], axis=1)
    w_ref[...] = jnp.concatenate(w_cols + [pad_f], axis=1)


def _topk(sim_T, local_id, patch_id):
    return pl.pallas_call(
        _topk_body,
        grid=(B,),
        in_specs=[
            pl.BlockSpec((K, N), lambda i: (0, i)),
            pl.BlockSpec((N, K), lambda i: (0, 0)),
            pl.BlockSpec((N, 1), lambda i: (0, 0)),
        ],
        out_specs=[
            pl.BlockSpec((N, KTOP), lambda i: (i, 0)),
            pl.BlockSpec((N, 32), lambda i: (i, 0)),
            pl.BlockSpec((N, 32), lambda i: (i, 0)),
        ],
        out_shape=[
            jax.ShapeDtypeStruct((ROWS, KTOP), _I32),
            jax.ShapeDtypeStruct((ROWS, 32), _I32),
            jax.ShapeDtypeStruct((ROWS, 32), _F32),
        ],
    )(sim_T, local_id, patch_id)


_HALF = 64


def _scatter_body(pos_ref, w_ref, z_ref, wt_ref, pbuf, wbuf, chunk, sem):
    wid = lax.axis_index("s") * NC + lax.axis_index("c")
    row0 = wid * RPW
    pltpu.sync_copy(pos_ref.at[pl.ds(row0, RPW)], pbuf)
    pltpu.sync_copy(w_ref.at[pl.ds(row0, RPW)], wbuf)
    for half in range(RPW // _HALF):
        pltpu.sync_copy(z_ref, chunk)

        def srow(i2, _):
            i = half * _HALF + i2
            rsplat = jnp.full((16,), i2, _I32)
            for j in range(2):
                pv = pbuf[i, pl.ds(16 * j, 16)]
                wv = wbuf[i, pl.ds(16 * j, 16)]
                plsc.addupdate_scatter(chunk, [rsplat, pv], wv,
                                       mask=wv > 0.0)
            return 0

        lax.fori_loop(0, _HALF, srow, 0)
        pltpu.sync_copy(chunk, wt_ref.at[pl.ds(row0 + half * _HALF, _HALF)])


def _scatter(sel_pos, sel_w, zeros_hbm):
    mesh = plsc.VectorSubcoreMesh(
        core_axis_name="c", subcore_axis_name="s",
        num_cores=NC, num_subcores=NS)
    fn = pl.kernel(
        _scatter_body,
        out_type=jax.ShapeDtypeStruct((ROWS, HW), _F32),
        mesh=mesh,
        compiler_params=pltpu.CompilerParams(needs_layout_passes=False),
        scratch_types=[
            pltpu.VMEM((RPW, 32), _I32),
            pltpu.VMEM((RPW, 32), _F32),
            pltpu.VMEM((_HALF, HW), _F32),
            pltpu.SemaphoreType.DMA,
        ],
    )
    return fn(sel_pos, sel_w, zeros_hbm)


def _mlp_body(wt_ref, f_ref, w1_ref, b1_ref, g1_ref, be1_ref, w2_ref,
              b2_ref, o_ref):
    xs = lax.dot_general(
        wt_ref[...], f_ref[...], (((1,), (0,)), ((), ())),
        preferred_element_type=_F32, precision=_HIGHEST) * (1.0 / 25.0)
    h1 = lax.dot_general(
        xs, w1_ref[...], (((1,), (0,)), ((), ())),
        preferred_element_type=_F32, precision=_HIGHEST) + b1_ref[...]
    h1 = (h1 / jnp.sqrt(jnp.float32(1.0 + 1e-5))) * g1_ref[...] + be1_ref[...]
    h1 = jnp.maximum(h1, 0.0)
    o_ref[...] = lax.dot_general(
        h1, w2_ref[...], (((1,), (0,)), ((), ())),
        preferred_element_type=_F32, precision=_HIGHEST) + b2_ref[...]


def _mlp(wt, feat_r, W1, b1, gamma1, beta1, W2, b2):
    return pl.pallas_call(
        _mlp_body,
        grid=(B,),
        in_specs=[
            pl.BlockSpec((None, N, HW), lambda i: (i, 0, 0)),
            pl.BlockSpec((None, HW, C), lambda i: (i, 0, 0)),
            pl.BlockSpec((C, 1024), lambda i: (0, 0)),
            pl.BlockSpec((1, 1024), lambda i: (0, 0)),
            pl.BlockSpec((1, 1024), lambda i: (0, 0)),
            pl.BlockSpec((1, 1024), lambda i: (0, 0)),
            pl.BlockSpec((1024, 256), lambda i: (0, 0)),
            pl.BlockSpec((1, 256), lambda i: (0, 0)),
        ],
        out_specs=pl.BlockSpec((None, N, 256), lambda i: (i, 0, 0)),
        out_shape=jax.ShapeDtypeStruct((B, N, 256), _F32),
    )(wt, feat_r, W1, b1, gamma1, beta1, W2, b2)


def kernel(patch_size, feats, num_patches, patch_ids, patch_local_ids,
           W1, b1, gamma1, beta1, W2, b2):
    feat = feats[0]
    feat_r = jnp.transpose(feat, (0, 2, 3, 1)).reshape(B, HW, C)
    patch_id = patch_ids[0][:, 0] if patch_ids.ndim == 3 else patch_ids[0]
    local_id = patch_local_ids[0]

    S = _gram(feat_r)
    sim_T = _simgather(S.reshape(B * HW, HW), patch_id.astype(_I32),
                       jnp.transpose(local_id.astype(_I32)))
    top_idx, sel_pos, sel_w = _topk(sim_T, local_id.astype(_I32),
                                    patch_id.astype(_I32).reshape(N, 1))
    zeros_hbm = jnp.zeros((_HALF, HW), _F32)
    wt = _scatter(sel_pos, sel_w, zeros_hbm)
    out = _mlp(wt.reshape(B, N, HW), feat_r, W1,
               b1.reshape(1, 1024), gamma1.reshape(1, 1024),
               beta1.reshape(1, 1024), W2, b2.reshape(1, 256))

    return (out.reshape(ROWS, 256), patch_id, local_id,
            top_idx.reshape(ROWS, KTOP, 1))


# speedup vs baseline: 24.7270x; 1.7559x over previous
import functools

import jax
import jax.numpy as jnp
from jax import lax
from jax.experimental import pallas as pl
from jax.experimental.pallas import tpu as pltpu
from jax.experimental.pallas import tpu_sc as plsc

B = 8
HW = 1024
C = 512
N = 512
K = 48
KTOP = 24
ROWS = B * N
NC, NS = 2, 16
NW = NC * NS
RPW = ROWS // NW

_F32 = jnp.float32
_I32 = jnp.int32
_HIGHEST = lax.Precision.HIGHEST


def _gram_body(f_ref, s_ref):
    x = f_ref[...]
    ss = jnp.sum(x * x, axis=1, keepdims=True)
    nrm = jnp.maximum(jnp.sqrt(ss), 1e-12)
    xn = x / nrm
    s_ref[...] = lax.dot_general(
        xn, xn, (((1,), (1,)), ((), ())),
        preferred_element_type=_F32)


def _gram(feat_r):
    return pl.pallas_call(
        _gram_body,
        grid=(B,),
        in_specs=[pl.BlockSpec((None, HW, C), lambda i: (i, 0, 0))],
        out_specs=pl.BlockSpec((None, HW, HW), lambda i: (i, 0, 0)),
        out_shape=jax.ShapeDtypeStruct((B, HW, HW), _F32),
    )(feat_r)


_SUBCH = 32


def _simgather_body(s_ref, pid_ref, lidT_ref, simT_ref,
                    lT, pref, idxb, rows_v, sbufT, sem):
    wid = lax.axis_index("s") * NC + lax.axis_index("c")
    b = wid // 4
    n0 = (wid % 4) * 128
    row0 = wid * RPW
    pltpu.sync_copy(lidT_ref.at[:, pl.ds(n0, 128)], lT)
    pltpu.sync_copy(pid_ref.at[pl.ds(n0, 128)], pref)
    iota16 = lax.iota(_I32, 16)

    for sc in range(RPW // _SUBCH):
        for h in range(2):
            idxb[pl.ds(16 * h, 16)] = pref[pl.ds(sc * _SUBCH + 16 * h, 16)] \
                + b * HW
        pltpu.async_copy(s_ref.at[idxb], rows_v, sem).wait()
        for g in range(2):
            maj = iota16 + g * 16

            def inner(j, _, _sc=sc, _g=g, _maj=maj):
                col = _sc * _SUBCH + _g * 16
                lvec = lT[j, pl.ds(col, 16)]
                sbufT[j, pl.ds(col, 16)] = plsc.load_gather(
                    rows_v, [_maj, lvec])
                return 0

            lax.fori_loop(0, K, inner, 0)
    pltpu.sync_copy(sbufT, simT_ref.at[:, pl.ds(row0, RPW)])


def _simgather(s_rows, patch_id, local_id_T):
    mesh = plsc.VectorSubcoreMesh(
        core_axis_name="c", subcore_axis_name="s",
        num_cores=NC, num_subcores=NS)
    fn = pl.kernel(
        _simgather_body,
        out_type=jax.ShapeDtypeStruct((K, ROWS), _F32),
        mesh=mesh,
        compiler_params=pltpu.CompilerParams(needs_layout_passes=False),
        scratch_types=[
            pltpu.VMEM((K, 128), _I32),
            pltpu.VMEM((128,), _I32),
            pltpu.VMEM((_SUBCH,), _I32),
            pltpu.VMEM((_SUBCH, HW), _F32),
            pltpu.VMEM((K, 128), _F32),
            pltpu.SemaphoreType.DMA,
        ],
    )
    return fn(s_rows, patch_id, local_id_T)


def _topk_body(sim_ref, locT_ref, pid_ref, top_ref, posT_ref):
    s = sim_ref[...]
    locT = locT_ref[...]
    si = lax.broadcasted_iota(_I32, (K, N), 0)
    rank = jnp.zeros((K, N), _I32)
    for j in range(K):
        sj = s[j:j + 1, :]
        beats = (sj > s) | ((sj == s) & (si > j))
        rank = rank + beats.astype(_I32)

    rows_top, rows_sel = [], []
    for r in range(KTOP):
        m = rank == r
        rows_top.append(jnp.sum(jnp.where(m, si, 0), axis=0, keepdims=True))
        rows_sel.append(jnp.sum(jnp.where(m, locT, 0), axis=0, keepdims=True))
    top_ref[...] = jnp.transpose(jnp.concatenate(rows_top, axis=0))
    posT_ref[...] = jnp.concatenate([pid_ref[...]] + rows_sel, axis=0)


def _topk(sim_T, local_id_T, patch_id):
    return pl.pallas_call(
        _topk_body,
        grid=(B,),
        in_specs=[
            pl.BlockSpec((K, N), lambda i: (0, i)),
            pl.BlockSpec((K, N), lambda i: (0, 0)),
            pl.BlockSpec((1, N), lambda i: (0, 0)),
        ],
        out_specs=[
            pl.BlockSpec((N, KTOP), lambda i: (i, 0)),
            pl.BlockSpec((KTOP + 1, N), lambda i: (0, i)),
        ],
        out_shape=[
            jax.ShapeDtypeStruct((ROWS, KTOP), _I32),
            jax.ShapeDtypeStruct((KTOP + 1, ROWS), _I32),
        ],
    )(sim_T, local_id_T, patch_id)


_HALF = 64


def _scatter_body(posT_ref, z_ref, wt_ref, pbuf, chunk, sem):
    wid = lax.axis_index("s") * NC + lax.axis_index("c")
    row0 = wid * RPW
    pltpu.sync_copy(posT_ref.at[:, pl.ds(row0, RPW)], pbuf)
    iota16 = lax.iota(_I32, 16)
    ones = jnp.ones((16,), _F32)
    for half in range(RPW // _HALF):
        pltpu.sync_copy(z_ref, chunk)
        for g in range(_HALF // 16):
            rvec = g * 16 + iota16
            for j in range(KTOP + 1):
                pv = pbuf[j, pl.ds(half * _HALF + g * 16, 16)]
                plsc.addupdate_scatter(chunk, [rvec, pv], ones)
        pltpu.sync_copy(chunk, wt_ref.at[pl.ds(row0 + half * _HALF, _HALF)])


def _scatter(pos_T, zeros_hbm):
    mesh = plsc.VectorSubcoreMesh(
        core_axis_name="c", subcore_axis_name="s",
        num_cores=NC, num_subcores=NS)
    fn = pl.kernel(
        _scatter_body,
        out_type=jax.ShapeDtypeStruct((ROWS, HW), _F32),
        mesh=mesh,
        compiler_params=pltpu.CompilerParams(needs_layout_passes=False),
        scratch_types=[
            pltpu.VMEM((KTOP + 1, RPW), _I32),
            pltpu.VMEM((_HALF, HW), _F32),
            pltpu.SemaphoreType.DMA,
        ],
    )
    return fn(pos_T, zeros_hbm)


def _mlp_body(wt_ref, f_ref, w1_ref, b1_ref, g1_ref, be1_ref, w2_ref,
              b2_ref, o_ref):
    xs = lax.dot_general(
        wt_ref[...], f_ref[...], (((1,), (0,)), ((), ())),
        preferred_element_type=_F32, precision=_HIGHEST) * (1.0 / 25.0)
    h1 = lax.dot_general(
        xs, w1_ref[...], (((1,), (0,)), ((), ())),
        preferred_element_type=_F32, precision=_HIGHEST) + b1_ref[...]
    h1 = (h1 / jnp.sqrt(jnp.float32(1.0 + 1e-5))) * g1_ref[...] + be1_ref[...]
    h1 = jnp.maximum(h1, 0.0)
    o_ref[...] = lax.dot_general(
        h1, w2_ref[...], (((1,), (0,)), ((), ())),
        preferred_element_type=_F32, precision=_HIGHEST) + b2_ref[...]


def _mlp(wt, feat_r, W1, b1, gamma1, beta1, W2, b2):
    return pl.pallas_call(
        _mlp_body,
        grid=(B,),
        in_specs=[
            pl.BlockSpec((None, N, HW), lambda i: (i, 0, 0)),
            pl.BlockSpec((None, HW, C), lambda i: (i, 0, 0)),
            pl.BlockSpec((C, 1024), lambda i: (0, 0)),
            pl.BlockSpec((1, 1024), lambda i: (0, 0)),
            pl.BlockSpec((1, 1024), lambda i: (0, 0)),
            pl.BlockSpec((1, 1024), lambda i: (0, 0)),
            pl.BlockSpec((1024, 256), lambda i: (0, 0)),
            pl.BlockSpec((1, 256), lambda i: (0, 0)),
        ],
        out_specs=pl.BlockSpec((None, N, 256), lambda i: (i, 0, 0)),
        out_shape=jax.ShapeDtypeStruct((B, N, 256), _F32),
    )(wt, feat_r, W1, b1, gamma1, beta1, W2, b2)


def kernel(patch_size, feats, num_patches, patch_ids, patch_local_ids,
           W1, b1, gamma1, beta1, W2, b2):
    feat = feats[0]
    feat_r = jnp.transpose(feat, (0, 2, 3, 1)).reshape(B, HW, C)
    patch_id = patch_ids[0][:, 0] if patch_ids.ndim == 3 else patch_ids[0]
    local_id = patch_local_ids[0]

    S = _gram(feat_r)
    local_T = jnp.transpose(local_id.astype(_I32))
    sim_T = _simgather(S.reshape(B * HW, HW), patch_id.astype(_I32), local_T)
    top_idx, pos_T = _topk(sim_T, local_T,
                           patch_id.astype(_I32).reshape(1, N))
    zeros_hbm = jnp.zeros((_HALF, HW), _F32)
    wt = _scatter(pos_T, zeros_hbm)
    out = _mlp(wt.reshape(B, N, HW), feat_r, W1,
               b1.reshape(1, 1024), gamma1.reshape(1, 1024),
               beta1.reshape(1, 1024), W2, b2.reshape(1, 256))

    return (out.reshape(ROWS, 256), patch_id, local_id,
            top_idx.reshape(ROWS, KTOP, 1))


# speedup vs baseline: 39.5671x; 1.6002x over previous
import functools

import jax
import jax.numpy as jnp
from jax import lax
from jax.experimental import pallas as pl
from jax.experimental.pallas import tpu as pltpu
from jax.experimental.pallas import tpu_sc as plsc

B = 8
HW = 1024
C = 512
N = 512
K = 48
KTOP = 24
ROWS = B * N
NC, NS = 2, 16
NW = NC * NS
RPW = ROWS // NW

_F32 = jnp.float32
_I32 = jnp.int32
_HIGHEST = lax.Precision.HIGHEST


def _gram_body(f_ref, s_ref):
    x = f_ref[...]
    ss = jnp.sum(x * x, axis=1, keepdims=True)
    nrm = jnp.maximum(jnp.sqrt(ss), 1e-12)
    xn = x / nrm
    s_ref[...] = lax.dot_general(
        xn, xn, (((1,), (1,)), ((), ())),
        preferred_element_type=_F32)


def _gram(feat_r):
    return pl.pallas_call(
        _gram_body,
        grid=(B,),
        in_specs=[pl.BlockSpec((None, HW, C), lambda i: (i, 0, 0))],
        out_specs=pl.BlockSpec((None, HW, HW), lambda i: (i, 0, 0)),
        out_shape=jax.ShapeDtypeStruct((B, HW, HW), _F32),
    )(feat_r)


_SUBCH = 32


def _simgather_body(s_ref, pid_ref, lidT_ref, simT_ref,
                    lT, pref, idxb, rows_v, sbufT, sem):
    wid = lax.axis_index("s") * NC + lax.axis_index("c")
    b = wid // 4
    n0 = (wid % 4) * 128
    row0 = wid * RPW
    pltpu.sync_copy(lidT_ref.at[:, pl.ds(n0, 128)], lT)
    pltpu.sync_copy(pid_ref.at[pl.ds(n0, 128)], pref)
    iota16 = lax.iota(_I32, 16)

    for sc in range(RPW // _SUBCH):
        for h in range(2):
            idxb[pl.ds(16 * h, 16)] = pref[pl.ds(sc * _SUBCH + 16 * h, 16)] \
                + b * HW
        pltpu.async_copy(s_ref.at[idxb], rows_v, sem).wait()
        for g in range(2):
            maj = iota16 + g * 16

            def inner(j, _, _sc=sc, _g=g, _maj=maj):
                col = _sc * _SUBCH + _g * 16
                lvec = lT[j, pl.ds(col, 16)]
                sbufT[j, pl.ds(col, 16)] = plsc.load_gather(
                    rows_v, [_maj, lvec])
                return 0

            lax.fori_loop(0, K, inner, 0)
    pltpu.sync_copy(sbufT, simT_ref.at[:, pl.ds(row0, RPW)])


def _simgather(s_rows, patch_id, local_id_T):
    mesh = plsc.VectorSubcoreMesh(
        core_axis_name="c", subcore_axis_name="s",
        num_cores=NC, num_subcores=NS)
    fn = pl.kernel(
        _simgather_body,
        out_type=jax.ShapeDtypeStruct((K, ROWS), _F32),
        mesh=mesh,
        compiler_params=pltpu.CompilerParams(needs_layout_passes=False),
        scratch_types=[
            pltpu.VMEM((K, 128), _I32),
            pltpu.VMEM((128,), _I32),
            pltpu.VMEM((_SUBCH,), _I32),
            pltpu.VMEM((_SUBCH, HW), _F32),
            pltpu.VMEM((K, 128), _F32),
            pltpu.SemaphoreType.DMA,
        ],
    )
    return fn(s_rows, patch_id, local_id_T)


def _topk_body(sim_ref, locT_ref, pid_ref, top_ref, posT_ref):
    s = sim_ref[...]
    locT = locT_ref[...]
    si = lax.broadcasted_iota(_I32, (K, N), 0)
    rank = jnp.zeros((K, N), _I32)
    for j in range(K):
        sj = s[j:j + 1, :]
        beats = (sj > s) | ((sj == s) & (si > j))
        rank = rank + beats.astype(_I32)

    rows_top, rows_sel = [], []
    for r in range(KTOP):
        m = rank == r
        rows_top.append(jnp.sum(jnp.where(m, si, 0), axis=0, keepdims=True))
        rows_sel.append(jnp.sum(jnp.where(m, locT, 0), axis=0, keepdims=True))
    top_ref[...] = jnp.transpose(jnp.concatenate(rows_top, axis=0))
    posT_ref[...] = jnp.concatenate([pid_ref[...]] + rows_sel, axis=0)


def _topk(sim_T, local_id_T, patch_id):
    return pl.pallas_call(
        _topk_body,
        grid=(B,),
        in_specs=[
            pl.BlockSpec((K, N), lambda i: (0, i)),
            pl.BlockSpec((K, N), lambda i: (0, 0)),
            pl.BlockSpec((1, N), lambda i: (0, 0)),
        ],
        out_specs=[
            pl.BlockSpec((N, KTOP), lambda i: (i, 0)),
            pl.BlockSpec((KTOP + 1, N), lambda i: (0, i)),
        ],
        out_shape=[
            jax.ShapeDtypeStruct((ROWS, KTOP), _I32),
            jax.ShapeDtypeStruct((KTOP + 1, ROWS), _I32),
        ],
    )(sim_T, local_id_T, patch_id)


_HALF = 64


def _scatter_body(posT_ref, z_ref, wt_ref, pbuf, chunk, sem):
    wid = lax.axis_index("s") * NC + lax.axis_index("c")
    row0 = wid * RPW
    pltpu.sync_copy(posT_ref.at[:, pl.ds(row0, RPW)], pbuf)
    iota16 = lax.iota(_I32, 16)
    ones = jnp.ones((16,), _F32)
    for half in range(RPW // _HALF):
        pltpu.sync_copy(z_ref, chunk)
        for g in range(_HALF // 16):
            rvec = g * 16 + iota16
            for j in range(KTOP + 1):
                pv = pbuf[j, pl.ds(half * _HALF + g * 16, 16)]
                plsc.addupdate_scatter(chunk, [rvec, pv], ones)
        pltpu.sync_copy(chunk, wt_ref.at[pl.ds(row0 + half * _HALF, _HALF)])


def _scatter(pos_T, zeros_hbm):
    mesh = plsc.VectorSubcoreMesh(
        core_axis_name="c", subcore_axis_name="s",
        num_cores=NC, num_subcores=NS)
    fn = pl.kernel(
        _scatter_body,
        out_type=jax.ShapeDtypeStruct((ROWS, HW), _F32),
        mesh=mesh,
        compiler_params=pltpu.CompilerParams(needs_layout_passes=False),
        scratch_types=[
            pltpu.VMEM((KTOP + 1, RPW), _I32),
            pltpu.VMEM((_HALF, HW), _F32),
            pltpu.SemaphoreType.DMA,
        ],
    )
    return fn(pos_T, zeros_hbm)


def _mlp_body(wt_ref, f_ref, w1_ref, b1_ref, g1_ref, be1_ref, w2_ref,
              b2_ref, o_ref):
    xs = lax.dot_general(
        wt_ref[...], f_ref[...], (((1,), (0,)), ((), ())),
        preferred_element_type=_F32) * (1.0 / 25.0)
    h1 = lax.dot_general(
        xs, w1_ref[...], (((1,), (0,)), ((), ())),
        preferred_element_type=_F32) + b1_ref[...]
    h1 = (h1 / jnp.sqrt(jnp.float32(1.0 + 1e-5))) * g1_ref[...] + be1_ref[...]
    h1 = jnp.maximum(h1, 0.0)
    o_ref[...] = lax.dot_general(
        h1, w2_ref[...], (((1,), (0,)), ((), ())),
        preferred_element_type=_F32) + b2_ref[...]


def _mlp(wt, feat_r, W1, b1, gamma1, beta1, W2, b2):
    return pl.pallas_call(
        _mlp_body,
        grid=(B,),
        in_specs=[
            pl.BlockSpec((None, N, HW), lambda i: (i, 0, 0)),
            pl.BlockSpec((None, HW, C), lambda i: (i, 0, 0)),
            pl.BlockSpec((C, 1024), lambda i: (0, 0)),
            pl.BlockSpec((1, 1024), lambda i: (0, 0)),
            pl.BlockSpec((1, 1024), lambda i: (0, 0)),
            pl.BlockSpec((1, 1024), lambda i: (0, 0)),
            pl.BlockSpec((1024, 256), lambda i: (0, 0)),
            pl.BlockSpec((1, 256), lambda i: (0, 0)),
        ],
        out_specs=pl.BlockSpec((None, N, 256), lambda i: (i, 0, 0)),
        out_shape=jax.ShapeDtypeStruct((B, N, 256), _F32),
    )(wt, feat_r, W1, b1, gamma1, beta1, W2, b2)


def kernel(patch_size, feats, num_patches, patch_ids, patch_local_ids,
           W1, b1, gamma1, beta1, W2, b2):
    feat = feats[0]
    feat_r = jnp.transpose(feat, (0, 2, 3, 1)).reshape(B, HW, C)
    patch_id = patch_ids[0][:, 0] if patch_ids.ndim == 3 else patch_ids[0]
    local_id = patch_local_ids[0]

    S = _gram(feat_r)
    local_T = jnp.transpose(local_id.astype(_I32))
    sim_T = _simgather(S.reshape(B * HW, HW), patch_id.astype(_I32), local_T)
    top_idx, pos_T = _topk(sim_T, local_T,
                           patch_id.astype(_I32).reshape(1, N))
    zeros_hbm = jnp.zeros((_HALF, HW), _F32)
    wt = _scatter(pos_T, zeros_hbm)
    out = _mlp(wt.reshape(B, N, HW), feat_r, W1,
               b1.reshape(1, 1024), gamma1.reshape(1, 1024),
               beta1.reshape(1, 1024), W2, b2.reshape(1, 256))

    return (out.reshape(ROWS, 256), patch_id, local_id,
            top_idx.reshape(ROWS, KTOP, 1))


# speedup vs baseline: 40.2767x; 1.0179x over previous
import functools

import jax
import jax.numpy as jnp
from jax import lax
from jax.experimental import pallas as pl
from jax.experimental.pallas import tpu as pltpu
from jax.experimental.pallas import tpu_sc as plsc

B = 8
HW = 1024
C = 512
N = 512
K = 48
KTOP = 24
ROWS = B * N
NC, NS = 2, 16
NW = NC * NS
RPW = ROWS // NW

_F32 = jnp.float32
_I32 = jnp.int32
_HIGHEST = lax.Precision.HIGHEST


def _gram_body(f_ref, s_ref):
    x = f_ref[...]
    ss = jnp.sum(x * x, axis=1, keepdims=True)
    nrm = jnp.maximum(jnp.sqrt(ss), 1e-12)
    xn = x / nrm
    s_ref[...] = lax.dot_general(
        xn, xn, (((1,), (1,)), ((), ())),
        preferred_element_type=_F32)


def _gram(feat_r):
    return pl.pallas_call(
        _gram_body,
        grid=(B,),
        in_specs=[pl.BlockSpec((None, HW, C), lambda i: (i, 0, 0))],
        out_specs=pl.BlockSpec((None, HW, HW), lambda i: (i, 0, 0)),
        out_shape=jax.ShapeDtypeStruct((B, HW, HW), _F32),
    )(feat_r)


_SUBCH = 32


def _simgather_body(s_ref, pid_ref, lidT_ref, simT_ref,
                    lT, pref, idxb, rows_v, sbufT, sem0, sem1):
    wid = lax.axis_index("s") * NC + lax.axis_index("c")
    b = wid // 4
    n0 = (wid % 4) * 128
    row0 = wid * RPW
    pltpu.sync_copy(lidT_ref.at[:, pl.ds(n0, 128)], lT)
    pltpu.sync_copy(pid_ref.at[pl.ds(n0, 128)], pref)
    iota16 = lax.iota(_I32, 16)
    sems = [sem0, sem1]
    nburst = RPW // _SUBCH

    def build(sc):
        for h in range(2):
            idxb[sc % 2, pl.ds(16 * h, 16)] = \
                pref[pl.ds(sc * _SUBCH + 16 * h, 16)] + b * HW

    def start(sc):
        return pltpu.async_copy(s_ref.at[idxb.at[sc % 2]],
                                rows_v.at[sc % 2], sems[sc % 2])

    build(0)
    descs = {0: start(0)}
    for sc in range(nburst):
        if sc + 1 < nburst:
            build(sc + 1)
            descs[sc + 1] = start(sc + 1)
        descs[sc].wait()
        for g in range(2):
            maj = iota16 + g * 16

            def inner(j, _, _sc=sc, _g=g, _maj=maj):
                col = _sc * _SUBCH + _g * 16
                lvec = lT[j, pl.ds(col, 16)]
                sbufT[j, pl.ds(col, 16)] = plsc.load_gather(
                    rows_v.at[_sc % 2], [_maj, lvec])
                return 0

            lax.fori_loop(0, K, inner, 0)
    pltpu.sync_copy(sbufT, simT_ref.at[:, pl.ds(row0, RPW)])


def _simgather(s_rows, patch_id, local_id_T):
    mesh = plsc.VectorSubcoreMesh(
        core_axis_name="c", subcore_axis_name="s",
        num_cores=NC, num_subcores=NS)
    fn = pl.kernel(
        _simgather_body,
        out_type=jax.ShapeDtypeStruct((K, ROWS), _F32),
        mesh=mesh,
        compiler_params=pltpu.CompilerParams(needs_layout_passes=False),
        scratch_types=[
            pltpu.VMEM((K, 128), _I32),
            pltpu.VMEM((128,), _I32),
            pltpu.VMEM((2, _SUBCH), _I32),
            pltpu.VMEM((2, _SUBCH, HW), _F32),
            pltpu.VMEM((K, 128), _F32),
            pltpu.SemaphoreType.DMA,
            pltpu.SemaphoreType.DMA,
        ],
    )
    return fn(s_rows, patch_id, local_id_T)


def _topk_body(sim_ref, locT_ref, pid_ref, top_ref, posT_ref):
    s = sim_ref[...]
    locT = locT_ref[...]
    si = lax.broadcasted_iota(_I32, (K, N), 0)
    rank = jnp.zeros((K, N), _I32)
    for j in range(K):
        sj = s[j:j + 1, :]
        beats = (sj > s) | ((sj == s) & (si > j))
        rank = rank + beats.astype(_I32)

    rows_top, rows_sel = [], []
    for r in range(KTOP):
        m = rank == r
        rows_top.append(jnp.sum(jnp.where(m, si, 0), axis=0, keepdims=True))
        rows_sel.append(jnp.sum(jnp.where(m, locT, 0), axis=0, keepdims=True))
    top_ref[...] = jnp.transpose(jnp.concatenate(rows_top, axis=0))
    posT_ref[...] = jnp.concatenate([pid_ref[...]] + rows_sel, axis=0)


def _topk(sim_T, local_id_T, patch_id):
    return pl.pallas_call(
        _topk_body,
        grid=(B,),
        in_specs=[
            pl.BlockSpec((K, N), lambda i: (0, i)),
            pl.BlockSpec((K, N), lambda i: (0, 0)),
            pl.BlockSpec((1, N), lambda i: (0, 0)),
        ],
        out_specs=[
            pl.BlockSpec((N, KTOP), lambda i: (i, 0)),
            pl.BlockSpec((KTOP + 1, N), lambda i: (0, i)),
        ],
        out_shape=[
            jax.ShapeDtypeStruct((ROWS, KTOP), _I32),
            jax.ShapeDtypeStruct((KTOP + 1, ROWS), _I32),
        ],
    )(sim_T, local_id_T, patch_id)


_HALF = 64


def _scatter_body(posT_ref, z_ref, wt_ref, pbuf, chunk, sem):
    wid = lax.axis_index("s") * NC + lax.axis_index("c")
    row0 = wid * RPW
    pltpu.sync_copy(posT_ref.at[:, pl.ds(row0, RPW)], pbuf)
    iota16 = lax.iota(_I32, 16)
    ones = jnp.ones((16,), _F32)
    for half in range(RPW // _HALF):
        pltpu.sync_copy(z_ref, chunk)
        for g in range(_HALF // 16):
            rvec = g * 16 + iota16
            for j in range(KTOP + 1):
                pv = pbuf[j, pl.ds(half * _HALF + g * 16, 16)]
                plsc.addupdate_scatter(chunk, [rvec, pv], ones)
        pltpu.sync_copy(chunk, wt_ref.at[pl.ds(row0 + half * _HALF, _HALF)])


def _scatter(pos_T, zeros_hbm):
    mesh = plsc.VectorSubcoreMesh(
        core_axis_name="c", subcore_axis_name="s",
        num_cores=NC, num_subcores=NS)
    fn = pl.kernel(
        _scatter_body,
        out_type=jax.ShapeDtypeStruct((ROWS, HW), _F32),
        mesh=mesh,
        compiler_params=pltpu.CompilerParams(needs_layout_passes=False),
        scratch_types=[
            pltpu.VMEM((KTOP + 1, RPW), _I32),
            pltpu.VMEM((_HALF, HW), _F32),
            pltpu.SemaphoreType.DMA,
        ],
    )
    return fn(pos_T, zeros_hbm)


def _mlp_body(wt_ref, f_ref, w1_ref, b1_ref, g1_ref, be1_ref, w2_ref,
              b2_ref, o_ref):
    xs = lax.dot_general(
        wt_ref[...], f_ref[...], (((1,), (0,)), ((), ())),
        preferred_element_type=_F32) * (1.0 / 25.0)
    h1 = lax.dot_general(
        xs, w1_ref[...], (((1,), (0,)), ((), ())),
        preferred_element_type=_F32) + b1_ref[...]
    h1 = (h1 / jnp.sqrt(jnp.float32(1.0 + 1e-5))) * g1_ref[...] + be1_ref[...]
    h1 = jnp.maximum(h1, 0.0)
    o_ref[...] = lax.dot_general(
        h1, w2_ref[...], (((1,), (0,)), ((), ())),
        preferred_element_type=_F32) + b2_ref[...]


def _mlp(wt, feat_r, W1, b1, gamma1, beta1, W2, b2):
    return pl.pallas_call(
        _mlp_body,
        grid=(B,),
        in_specs=[
            pl.BlockSpec((None, N, HW), lambda i: (i, 0, 0)),
            pl.BlockSpec((None, HW, C), lambda i: (i, 0, 0)),
            pl.BlockSpec((C, 1024), lambda i: (0, 0)),
            pl.BlockSpec((1, 1024), lambda i: (0, 0)),
            pl.BlockSpec((1, 1024), lambda i: (0, 0)),
            pl.BlockSpec((1, 1024), lambda i: (0, 0)),
            pl.BlockSpec((1024, 256), lambda i: (0, 0)),
            pl.BlockSpec((1, 256), lambda i: (0, 0)),
        ],
        out_specs=pl.BlockSpec((None, N, 256), lambda i: (i, 0, 0)),
        out_shape=jax.ShapeDtypeStruct((B, N, 256), _F32),
    )(wt, feat_r, W1, b1, gamma1, beta1, W2, b2)


def kernel(patch_size, feats, num_patches, patch_ids, patch_local_ids,
           W1, b1, gamma1, beta1, W2, b2):
    feat = feats[0]
    feat_r = jnp.transpose(feat, (0, 2, 3, 1)).reshape(B, HW, C)
    patch_id = patch_ids[0][:, 0] if patch_ids.ndim == 3 else patch_ids[0]
    local_id = patch_local_ids[0]

    S = _gram(feat_r)
    local_T = jnp.transpose(local_id.astype(_I32))
    sim_T = _simgather(S.reshape(B * HW, HW), patch_id.astype(_I32), local_T)
    top_idx, pos_T = _topk(sim_T, local_T,
                           patch_id.astype(_I32).reshape(1, N))
    zeros_hbm = jnp.zeros((_HALF, HW), _F32)
    wt = _scatter(pos_T, zeros_hbm)
    out = _mlp(wt.reshape(B, N, HW), feat_r, W1,
               b1.reshape(1, 1024), gamma1.reshape(1, 1024),
               beta1.reshape(1, 1024), W2, b2.reshape(1, 256))

    return (out.reshape(ROWS, 256), patch_id, local_id,
            top_idx.reshape(ROWS, KTOP, 1))
